# Initial kernel scaffold; baseline (speedup 1.0000x reference)
import jax
import jax.numpy as jnp
from jax.experimental import pallas as pl


def kernel(z, embed_weight):
    raise NotImplementedError("write your pallas kernel here")


# speedup vs baseline: 10.5131x; 10.5131x over previous
import functools

import jax
import jax.numpy as jnp
from jax import lax
from jax.experimental import pallas as pl
from jax.experimental.pallas import tpu as pltpu
from jax.experimental.pallas import tpu_sc as plsc

N_CODES = 8192
E_DIM = 256
CODE_TILE = 1024


def _dist_argmin_body(z_ref, e_ref, idx_ref):
    zb = z_ref[0]
    n_tok = zb.shape[1]
    zsq = jnp.sum(zb * zb, axis=0)[None, :]
    def body(k, carry):
        best_v, best_i = carry
        et = e_ref[pl.ds(k * CODE_TILE, CODE_TILE), :]
        m = lax.dot_general(et, zb, (((1,), (0,)), ((), ())),
                            preferred_element_type=jnp.float32)
        esq = jnp.sum(et * et, axis=1)[:, None]
        d = (zsq + esq) - 2.0 * m
        tmin = jnp.min(d, axis=0)[None, :]
        rows = lax.broadcasted_iota(jnp.int32, d.shape, 0)
        tidx = (jnp.min(jnp.where(d == tmin, rows, CODE_TILE), axis=0)
                + k * CODE_TILE)
        tminv = tmin[0]
        upd = tminv < best_v
        return jnp.where(upd, tminv, best_v), jnp.where(upd, tidx, best_i)

    init = (jnp.full((n_tok,), jnp.inf, jnp.float32),
            jnp.zeros((n_tok,), jnp.int32))
    _, best_i = lax.fori_loop(0, N_CODES // CODE_TILE, body, init)
    idx_ref[0, 0] = best_i


def _dist_argmin(zb3, embed):
    b, c, n_tok = zb3.shape
    return pl.pallas_call(
        _dist_argmin_body,
        grid=(b,),
        in_specs=[pl.BlockSpec((1, c, n_tok), lambda i: (i, 0, 0)),
                  pl.BlockSpec((N_CODES, E_DIM), lambda i: (0, 0))],
        out_specs=pl.BlockSpec((1, 1, n_tok), lambda i: (i, 0, 0)),
        out_shape=jax.ShapeDtypeStruct((b, 1, n_tok), jnp.int32),
    )(zb3, embed)


def _gather_rows(table, idx):
    n_rows, d = table.shape
    n_tok = idx.shape[0]
    num_workers = 32
    bpw = n_tok // num_workers
    mesh = plsc.VectorSubcoreMesh(core_axis_name="c", subcore_axis_name="s")

    @functools.partial(
        pl.kernel, mesh=mesh,
        out_type=jax.ShapeDtypeStruct((n_tok, d), jnp.float32),
        scratch_types=[pltpu.VMEM((bpw,), jnp.int32),
                       pltpu.VMEM((bpw, d), jnp.float32),
                       pltpu.SemaphoreType.DMA],
    )
    def gather_kernel(table_hbm, idx_hbm, out_hbm, idx_v, rows_v, sem):
        wid = lax.axis_index("s") * 2 + lax.axis_index("c")
        base = wid * bpw
        pltpu.sync_copy(idx_hbm.at[pl.ds(base, bpw)], idx_v)
        pltpu.async_copy(table_hbm.at[idx_v], rows_v, sem).wait()
        pltpu.sync_copy(rows_v, out_hbm.at[pl.ds(base, bpw)])

    return gather_kernel(table, idx)


def kernel(z, embed_weight):
    b, c, h, w = z.shape
    zb3 = z.reshape(b, c, h * w)
    idx3 = _dist_argmin(zb3, embed_weight)
    idx_flat = idx3.reshape(-1)
    zq_flat = _gather_rows(embed_weight, idx_flat)
    z_q = zq_flat.reshape(b, h, w, c).transpose(0, 3, 1, 2)
    index = idx3.reshape(b, h, w)
    loss = jnp.zeros((), z.dtype)
    return z_q, index, loss
